# baseline (speedup 1.0000x reference)
import functools

import jax
import jax.numpy as jnp
import numpy as np
from jax import lax
from jax.experimental import pallas as pl
from jax.experimental.pallas import tpu as pltpu
from jax.experimental.pallas import tpu_sc as plsc

NC, NS, L = 2, 16, 16
NW = NC * NS
B, S, D = 1024, 200, 64
ROWS = B * S
RPW = ROWS // NW
CH = 100
HALVES = S // CH
NSEQ = RPW // S
VECS = D // L
SCALE = float(np.sqrt(np.float32(D)))

_mesh = plsc.VectorSubcoreMesh(core_axis_name="c", subcore_axis_name="s")


@functools.partial(
    pl.kernel,
    out_type=jax.ShapeDtypeStruct((ROWS // CH, CH, D), jnp.float32),
    mesh=_mesh,
    scratch_types=[
        pltpu.VMEM((NSEQ, HALVES, CH), jnp.int32),
        pltpu.VMEM((S, D), jnp.float32),
        pltpu.VMEM((HALVES, CH, D), jnp.float32),
        pltpu.VMEM((HALVES, CH, D), jnp.float32),
        pltpu.SemaphoreType.DMA,
        pltpu.SemaphoreType.DMA,
    ],
    compiler_params=pltpu.CompilerParams(use_tc_tiling_on_sc=False),
)
def _embed_sc(token_hbm, word_hbm, pos_hbm, out_hbm,
              idx_v, pos_v, buf_a, buf_b, sem_a, sem_b):
    wid = lax.axis_index("s") * NC + lax.axis_index("c")
    seq0 = wid * NSEQ

    pltpu.sync_copy(token_hbm.at[wid], idx_v)
    pltpu.sync_copy(pos_hbm, pos_v)

    def gather(seq, buf, sem):
        for h in range(HALVES):
            pltpu.async_copy(word_hbm.at[idx_v.at[seq, h]], buf.at[h], sem)

    def wait(seq, buf, sem):
        for h in range(HALVES):
            pltpu.make_async_copy(word_hbm.at[idx_v.at[seq, h]], buf.at[h],
                                  sem).wait()

    def fma_rows(buf):
        for h in range(HALVES):
            def row(r, _):
                for c in range(VECS):
                    sl = pl.ds(c * L, L)
                    buf[h, r, sl] = buf[h, r, sl] * SCALE + pos_v[h * CH + r, sl]
                return ()
            lax.fori_loop(0, CH, row, (), unroll=2)

    def store(seq, buf):
        pltpu.sync_copy(buf, out_hbm.at[pl.ds((seq0 + seq) * HALVES, HALVES)])

    gather(0, buf_a, sem_a)
    gather(1, buf_b, sem_b)

    def body(i, _):
        j = 2 * i
        wait(j, buf_a, sem_a)
        fma_rows(buf_a)
        store(j, buf_a)

        @pl.when(j + 2 < NSEQ)
        def _():
            gather(j + 2, buf_a, sem_a)

        wait(j + 1, buf_b, sem_b)
        fma_rows(buf_b)
        store(j + 1, buf_b)

        @pl.when(j + 3 < NSEQ)
        def _():
            gather(j + 3, buf_b, sem_b)

        return ()

    lax.fori_loop(0, NSEQ // 2, body, ())


def kernel(token, word_table, pos_table):
    tok = token.reshape(NW, NSEQ, HALVES, CH).astype(jnp.int32)
    out = _embed_sc(tok, word_table, pos_table)
    return out.reshape(B, S, D)


# speedup vs baseline: 1.0016x; 1.0016x over previous
import functools

import jax
import jax.numpy as jnp
import numpy as np
from jax import lax
from jax.experimental import pallas as pl
from jax.experimental.pallas import tpu as pltpu
from jax.experimental.pallas import tpu_sc as plsc

NC, NS, L = 2, 16, 16
NW = NC * NS
B, S, D = 1024, 200, 64
CH = 100
HALVES = S // CH
NSEQ = B // NW
VECS = D // L
SCALE = float(np.sqrt(np.float32(D)))

_mesh = plsc.VectorSubcoreMesh(core_axis_name="c", subcore_axis_name="s")


@functools.partial(
    pl.kernel,
    out_type=jax.ShapeDtypeStruct((B, S, D), jnp.float32),
    mesh=_mesh,
    scratch_types=[
        pltpu.VMEM((NSEQ, HALVES, CH), jnp.int32),
        pltpu.VMEM((S, D), jnp.float32),
        pltpu.VMEM((S, D), jnp.float32),
        pltpu.VMEM((S, D), jnp.float32),
        pltpu.SemaphoreType.DMA,
        pltpu.SemaphoreType.DMA,
    ],
    compiler_params=pltpu.CompilerParams(use_tc_tiling_on_sc=False),
)
def _embed_sc(token_hbm, word_hbm, pos_hbm, out_hbm,
              idx_v, pos_v, buf_a, buf_b, sem_a, sem_b):
    wid = lax.axis_index("s") * NC + lax.axis_index("c")
    b0 = wid * NSEQ

    pltpu.sync_copy(token_hbm.at[wid], idx_v)
    pltpu.sync_copy(pos_hbm, pos_v)

    def gather(seq, buf, sem):
        for h in range(HALVES):
            pltpu.async_copy(word_hbm.at[idx_v.at[seq, h]],
                             buf.at[pl.ds(h * CH, CH)], sem)

    def wait(seq, buf, sem):
        for h in range(HALVES):
            pltpu.make_async_copy(word_hbm.at[idx_v.at[seq, h]],
                                  buf.at[pl.ds(h * CH, CH)], sem).wait()

    def fma_rows(buf):
        def row(r, _):
            for c in range(VECS):
                sl = pl.ds(c * L, L)
                buf[r, sl] = buf[r, sl] * SCALE + pos_v[r, sl]
            return ()
        lax.fori_loop(0, S, row, (), unroll=2)

    def store(seq, buf):
        pltpu.sync_copy(buf, out_hbm.at[b0 + seq])

    gather(0, buf_a, sem_a)
    gather(1, buf_b, sem_b)

    def body(i, _):
        j = 2 * i
        wait(j, buf_a, sem_a)
        fma_rows(buf_a)
        store(j, buf_a)

        @pl.when(j + 2 < NSEQ)
        def _():
            gather(j + 2, buf_a, sem_a)

        wait(j + 1, buf_b, sem_b)
        fma_rows(buf_b)
        store(j + 1, buf_b)

        @pl.when(j + 3 < NSEQ)
        def _():
            gather(j + 3, buf_b, sem_b)

        return ()

    lax.fori_loop(0, NSEQ // 2, body, ())


def kernel(token, word_table, pos_table):
    tok = token.reshape(NW, NSEQ, HALVES, CH).astype(jnp.int32)
    return _embed_sc(tok, word_table, pos_table)
